# Initial kernel scaffold; baseline (speedup 1.0000x reference)
import jax
import jax.numpy as jnp
from jax.experimental import pallas as pl


def kernel(node_features, edge_features, senders, receivers, params):
    raise NotImplementedError("write your pallas kernel here")


# speedup vs baseline: 3.5407x; 3.5407x over previous
import functools

import jax
import jax.numpy as jnp
from jax import lax
from jax.experimental import pallas as pl
from jax.experimental.pallas import tpu as pltpu
from jax.experimental.pallas import tpu_sc as plsc

LAT = 128
NN = 10000
NE = 320000
NCORE = 2
NSUB = 16
NW = NCORE * NSUB
EPW = NE // NW
CH = 80
NCH = EPW // CH
NNP = 10240
NPS = NNP // NSUB


def _sc_gather(node_lat, senders, receivers):
    mesh = plsc.VectorSubcoreMesh(core_axis_name="c", subcore_axis_name="s")

    @functools.partial(
        pl.kernel,
        mesh=mesh,
        out_type=[
            jax.ShapeDtypeStruct((NE, LAT), jnp.float32),
            jax.ShapeDtypeStruct((NE, LAT), jnp.float32),
        ],
        scratch_types=[
            pltpu.VMEM((EPW,), jnp.int32),
            pltpu.VMEM((EPW,), jnp.int32),
            pltpu.VMEM((CH, LAT), jnp.float32),
            pltpu.VMEM((CH, LAT), jnp.float32),
            pltpu.SemaphoreType.DMA,
            pltpu.SemaphoreType.DMA,
        ],
    )
    def k(node_hbm, snd_hbm, rcv_hbm, out_s, out_r,
          idx_s, idx_r, buf_s, buf_r, sem_s, sem_r):
        wid = lax.axis_index("c") * NSUB + lax.axis_index("s")
        base = wid * EPW
        pltpu.sync_copy(snd_hbm.at[pl.ds(base, EPW)], idx_s)
        pltpu.sync_copy(rcv_hbm.at[pl.ds(base, EPW)], idx_r)

        def body(j, carry):
            off = j * CH
            cs = pltpu.async_copy(node_hbm.at[idx_s.at[pl.ds(off, CH)]],
                                  buf_s, sem_s)
            cr = pltpu.async_copy(node_hbm.at[idx_r.at[pl.ds(off, CH)]],
                                  buf_r, sem_r)
            cs.wait()
            cr.wait()
            pltpu.sync_copy(buf_s, out_s.at[pl.ds(base + off, CH)])
            pltpu.sync_copy(buf_r, out_r.at[pl.ds(base + off, CH)])
            return carry

        lax.fori_loop(0, NCH, body, 0)

    return k(node_lat, senders, receivers)


def _sc_scatter(edge_lat, recv3, zeros):
    mesh = plsc.VectorSubcoreMesh(core_axis_name="c", subcore_axis_name="s")

    @functools.partial(
        pl.kernel,
        mesh=mesh,
        out_type=jax.ShapeDtypeStruct((NCORE, NNP, LAT), jnp.float32),
        scratch_types=[
            pltpu.VMEM((NCH, CH), jnp.int32),
            pltpu.VMEM((CH, LAT), jnp.float32),
            pltpu.VMEM_SHARED((NNP, LAT), jnp.float32),
            pltpu.SemaphoreType.DMA,
        ],
    )
    def k(edge_hbm, idx_hbm, zero_hbm, out, idxv, rows, agg, sem):
        c = lax.axis_index("c")
        s = lax.axis_index("s")
        wid = c * NSUB + s
        base = wid * EPW
        pltpu.sync_copy(idx_hbm.at[wid], idxv)

        @pl.when(s == 0)
        def _():
            pltpu.sync_copy(zero_hbm, agg)

        plsc.subcore_barrier()

        def body(j, carry):
            pltpu.sync_copy(edge_hbm.at[pl.ds(base + j * CH, CH)], rows)
            pltpu.sync_copy(rows, agg.at[idxv.at[j]], add=True)
            return carry

        lax.fori_loop(0, NCH, body, 0)
        plsc.subcore_barrier()
        pltpu.sync_copy(agg.at[pl.ds(s * NPS, NPS)],
                        out.at[c, pl.ds(s * NPS, NPS)])

    return k(edge_lat, recv3, zeros)


def _ln(h, g, beta):
    mu = jnp.mean(h, axis=-1, keepdims=True)
    d = h - mu
    var = jnp.mean(d * d, axis=-1, keepdims=True)
    return d * lax.rsqrt(var + 1e-5) * g + beta


def _dot(a, b):
    return jnp.dot(a, b, preferred_element_type=jnp.float32)


def _full(shape):
    nd = len(shape)
    return pl.BlockSpec(shape, lambda i: (0,) * nd)


def _enc_call(x, p, block):
    n, din = x.shape

    def body(x_ref, w0, b0, w1, b1, w2, b2, g, beta, o_ref):
        h = jnp.maximum(_dot(x_ref[...], w0[...]) + b0[...], 0.0)
        h = jnp.maximum(_dot(h, w1[...]) + b1[...], 0.0)
        h = _dot(h, w2[...]) + b2[...]
        o_ref[...] = _ln(h, g[...], beta[...])

    args = (x, p['W0'], p['b0'].reshape(1, -1), p['W1'],
            p['b1'].reshape(1, -1), p['W2'], p['b2'].reshape(1, -1),
            p['g'].reshape(1, -1), p['beta'].reshape(1, -1))
    in_specs = [pl.BlockSpec((block, din), lambda i: (i, 0))]
    in_specs += [_full(a.shape) for a in args[1:]]
    return pl.pallas_call(
        body,
        grid=(n // block,),
        in_specs=in_specs,
        out_specs=pl.BlockSpec((block, LAT), lambda i: (i, 0)),
        out_shape=jax.ShapeDtypeStruct((n, LAT), jnp.float32),
    )(*args)


def _edge_step(edge_lat, sf, rf, p, block=2560):
    w0e = p['W0'][0:LAT]
    w0s = p['W0'][LAT:2 * LAT]
    w0r = p['W0'][2 * LAT:3 * LAT]

    def body(e_ref, s_ref, r_ref, w0e_, w0s_, w0r_, b0, w1, b1, w2, b2,
             g, beta, o_ref):
        e = e_ref[...]
        h = (_dot(e, w0e_[...]) + _dot(s_ref[...], w0s_[...])
             + _dot(r_ref[...], w0r_[...]) + b0[...])
        h = jnp.maximum(h, 0.0)
        h = jnp.maximum(_dot(h, w1[...]) + b1[...], 0.0)
        h = _dot(h, w2[...]) + b2[...]
        o_ref[...] = e + _ln(h, g[...], beta[...])

    args = (edge_lat, sf, rf, w0e, w0s, w0r, p['b0'].reshape(1, -1),
            p['W1'], p['b1'].reshape(1, -1), p['W2'], p['b2'].reshape(1, -1),
            p['g'].reshape(1, -1), p['beta'].reshape(1, -1))
    row_spec = pl.BlockSpec((block, LAT), lambda i: (i, 0))
    in_specs = [row_spec, row_spec, row_spec]
    in_specs += [_full(a.shape) for a in args[3:]]
    return pl.pallas_call(
        body,
        grid=(NE // block,),
        in_specs=in_specs,
        out_specs=row_spec,
        out_shape=jax.ShapeDtypeStruct((NE, LAT), jnp.float32),
    )(*args)


def _node_step(node_lat, agg2, p, block=2000):
    w0n = p['W0'][0:LAT]
    w0a = p['W0'][LAT:2 * LAT]

    def body(n_ref, a_ref, w0n_, w0a_, b0, w1, b1, w2, b2, g, beta, o_ref):
        nl = n_ref[...]
        agg = a_ref[0] + a_ref[1]
        h = _dot(nl, w0n_[...]) + _dot(agg, w0a_[...]) + b0[...]
        h = jnp.maximum(h, 0.0)
        h = jnp.maximum(_dot(h, w1[...]) + b1[...], 0.0)
        h = _dot(h, w2[...]) + b2[...]
        o_ref[...] = nl + _ln(h, g[...], beta[...])

    args = (node_lat, agg2, w0n, w0a, p['b0'].reshape(1, -1), p['W1'],
            p['b1'].reshape(1, -1), p['W2'], p['b2'].reshape(1, -1),
            p['g'].reshape(1, -1), p['beta'].reshape(1, -1))
    in_specs = [pl.BlockSpec((block, LAT), lambda i: (i, 0)),
                pl.BlockSpec((NCORE, block, LAT), lambda i: (0, i, 0))]
    in_specs += [_full(a.shape) for a in args[2:]]
    return pl.pallas_call(
        body,
        grid=(NN // block,),
        in_specs=in_specs,
        out_specs=pl.BlockSpec((block, LAT), lambda i: (i, 0)),
        out_shape=jax.ShapeDtypeStruct((NN, LAT), jnp.float32),
    )(*args)


def _decode(node_lat, p, block=2000):
    def body(x_ref, w0, b0, w1, b1, w2, b2, o_ref):
        h = jnp.maximum(_dot(x_ref[...], w0[...]) + b0[...], 0.0)
        h = jnp.maximum(_dot(h, w1[...]) + b1[...], 0.0)
        o_ref[...] = _dot(h, w2[...]) + b2[...]

    dout = p['W2'].shape[1]
    args = (node_lat, p['W0'], p['b0'].reshape(1, -1), p['W1'],
            p['b1'].reshape(1, -1), p['W2'], p['b2'].reshape(1, -1))
    in_specs = [pl.BlockSpec((block, LAT), lambda i: (i, 0))]
    in_specs += [_full(a.shape) for a in args[1:]]
    return pl.pallas_call(
        body,
        grid=(NN // block,),
        in_specs=in_specs,
        out_specs=pl.BlockSpec((block, dout), lambda i: (i, 0)),
        out_shape=jax.ShapeDtypeStruct((NN, dout), jnp.float32),
    )(*args)


def kernel(node_features, edge_features, senders, receivers, params):
    node_lat = _enc_call(node_features, params['node_enc'], block=2000)
    edge_lat = _enc_call(edge_features, params['edge_enc'], block=2560)
    recv3 = receivers.reshape(NW, NCH, CH)
    zeros = jnp.zeros((NNP, LAT), jnp.float32)
    for step in params['steps']:
        sf, rf = _sc_gather(node_lat, senders, receivers)
        edge_lat = _edge_step(edge_lat, sf, rf, step['edge'])
        agg2 = _sc_scatter(edge_lat, recv3, zeros)
        node_lat = _node_step(node_lat, agg2, step['node'])
    return _decode(node_lat, params['decoder'])


# speedup vs baseline: 4.7436x; 1.3397x over previous
import functools

import jax
import jax.numpy as jnp
from jax import lax
from jax.experimental import pallas as pl
from jax.experimental.pallas import tpu as pltpu
from jax.experimental.pallas import tpu_sc as plsc

LAT = 128
NN = 10000
NE = 320000
NCORE = 2
NSUB = 16
NW = NCORE * NSUB
EPW = NE // NW
CH = 80
NCH = EPW // CH
NSLOT = 5
NSLOT_SC = 3
NNP = 10240
NPS = NNP // NSUB


def _sc_gather_sum(sp, rp, senders, receivers):
    mesh = plsc.VectorSubcoreMesh(core_axis_name="c", subcore_axis_name="s")
    scr = [pltpu.VMEM((EPW,), jnp.int32), pltpu.VMEM((EPW,), jnp.int32)]
    scr += [pltpu.VMEM((CH, LAT), jnp.float32) for _ in range(2 * NSLOT)]
    scr += [pltpu.SemaphoreType.DMA for _ in range(2 * NSLOT + 1)]

    @functools.partial(
        pl.kernel,
        mesh=mesh,
        out_type=jax.ShapeDtypeStruct((NE, LAT), jnp.float32),
        scratch_types=scr,
    )
    def k(sp_hbm, rp_hbm, snd_hbm, rcv_hbm, out, idx_s, idx_r, *rest):
        bufs = rest[:2 * NSLOT]
        sems = rest[2 * NSLOT:]
        bs = bufs[0::2]
        br = bufs[1::2]
        sg = sems[:NSLOT]
        rg = sems[NSLOT:2 * NSLOT]
        ws = sems[2 * NSLOT]
        wid = lax.axis_index("c") * NSUB + lax.axis_index("s")
        base = wid * EPW
        pltpu.sync_copy(snd_hbm.at[pl.ds(base, EPW)], idx_s)
        pltpu.sync_copy(rcv_hbm.at[pl.ds(base, EPW)], idx_r)

        def body(kk, carry):
            j0 = kk * NSLOT
            gs, gr = [], []
            for t in range(NSLOT):
                off = (j0 + t) * CH
                gs.append(pltpu.async_copy(
                    sp_hbm.at[idx_s.at[pl.ds(off, CH)]], bs[t], sg[t]))
                gr.append(pltpu.async_copy(
                    rp_hbm.at[idx_r.at[pl.ds(off, CH)]], br[t], rg[t]))
            wcs = []
            for t in range(NSLOT):
                gs[t].wait()
                gr[t].wait()

                def add_row(r, _, t=t):
                    for cc in range(LAT // 16):
                        sl = pl.ds(cc * 16, 16)
                        bs[t][r, sl] = bs[t][r, sl] + br[t][r, sl]
                    return 0

                lax.fori_loop(0, CH, add_row, 0)
                wcs.append(pltpu.async_copy(
                    bs[t], out.at[pl.ds(base + (j0 + t) * CH, CH)], ws))
            for w in wcs:
                w.wait()
            return carry

        lax.fori_loop(0, NCH // NSLOT, body, 0)

    return k(sp, rp, senders, receivers)


def _sc_scatter(edge_lat, recv3, zeros):
    mesh = plsc.VectorSubcoreMesh(core_axis_name="c", subcore_axis_name="s")
    scr = [pltpu.VMEM((NCH, CH), jnp.int32)]
    scr += [pltpu.VMEM((CH, LAT), jnp.float32) for _ in range(NSLOT_SC)]
    scr += [pltpu.VMEM_SHARED((NNP, LAT), jnp.float32)]
    scr += [pltpu.SemaphoreType.DMA for _ in range(NSLOT_SC + 1)]

    @functools.partial(
        pl.kernel,
        mesh=mesh,
        out_type=jax.ShapeDtypeStruct((NCORE, NNP, LAT), jnp.float32),
        scratch_types=scr,
    )
    def k(edge_hbm, idx_hbm, zero_hbm, out, idxv, *rest):
        rows = rest[:NSLOT_SC]
        agg = rest[NSLOT_SC]
        sems = rest[NSLOT_SC + 1:]
        ld = sems[:NSLOT_SC]
        sc = sems[NSLOT_SC]
        c = lax.axis_index("c")
        s = lax.axis_index("s")
        wid = c * NSUB + s
        base = wid * EPW
        pltpu.sync_copy(idx_hbm.at[wid], idxv)
        pltpu.sync_copy(zero_hbm.at[pl.ds(s * NPS, NPS)],
                        agg.at[pl.ds(s * NPS, NPS)])
        plsc.subcore_barrier()

        def body(kk, carry):
            j0 = kk * NSLOT_SC
            lds = [pltpu.async_copy(
                edge_hbm.at[pl.ds(base + (j0 + t) * CH, CH)], rows[t], ld[t])
                for t in range(NSLOT_SC)]
            scs = []
            for t in range(NSLOT_SC):
                lds[t].wait()
                scs.append(pltpu.async_copy(
                    rows[t], agg.at[idxv.at[j0 + t]], sc, add=True))
            for x in scs:
                x.wait()
            return carry

        n_body = NCH // NSLOT_SC
        lax.fori_loop(0, n_body, body, 0)
        for t in range(NCH - n_body * NSLOT_SC):
            j = n_body * NSLOT_SC + t
            pltpu.sync_copy(edge_hbm.at[pl.ds(base + j * CH, CH)], rows[t])
            pltpu.async_copy(rows[t], agg.at[idxv.at[j]], sc,
                             add=True).wait()
        plsc.subcore_barrier()
        pltpu.sync_copy(agg.at[pl.ds(s * NPS, NPS)],
                        out.at[c, pl.ds(s * NPS, NPS)])

    return k(edge_lat, recv3, zeros)


def _ln(h, g, beta):
    mu = jnp.mean(h, axis=-1, keepdims=True)
    d = h - mu
    var = jnp.mean(d * d, axis=-1, keepdims=True)
    return d * lax.rsqrt(var + 1e-5) * g + beta


def _dot(a, b):
    return jnp.dot(a, b, preferred_element_type=jnp.float32)


def _full(shape):
    nd = len(shape)
    return pl.BlockSpec(shape, lambda i: (0,) * nd)


def _enc_call(x, p, block, proj=None):
    n, din = x.shape

    def body(x_ref, w0, b0, w1, b1, w2, b2, g, beta, *rest):
        h = jnp.maximum(_dot(x_ref[...], w0[...]) + b0[...], 0.0)
        h = jnp.maximum(_dot(h, w1[...]) + b1[...], 0.0)
        h = _dot(h, w2[...]) + b2[...]
        y = _ln(h, g[...], beta[...])
        if proj is None:
            rest[-1][...] = y
        else:
            ws, wr, o_ref, sp_ref, rp_ref = rest
            o_ref[...] = y
            sp_ref[...] = _dot(y, ws[...])
            rp_ref[...] = _dot(y, wr[...])

    args = [x, p['W0'], p['b0'].reshape(1, -1), p['W1'],
            p['b1'].reshape(1, -1), p['W2'], p['b2'].reshape(1, -1),
            p['g'].reshape(1, -1), p['beta'].reshape(1, -1)]
    if proj is not None:
        args += [proj[0], proj[1]]
    in_specs = [pl.BlockSpec((block, din), lambda i: (i, 0))]
    in_specs += [_full(a.shape) for a in args[1:]]
    row_spec = pl.BlockSpec((block, LAT), lambda i: (i, 0))
    n_out = 1 if proj is None else 3
    return pl.pallas_call(
        body,
        grid=(n // block,),
        in_specs=in_specs,
        out_specs=[row_spec] * n_out if proj else row_spec,
        out_shape=([jax.ShapeDtypeStruct((n, LAT), jnp.float32)] * n_out
                   if proj else jax.ShapeDtypeStruct((n, LAT), jnp.float32)),
    )(*args)


def _edge_step(edge_lat, gsum, p, block=2560):
    w0e = p['W0'][0:LAT]

    def body(e_ref, g_ref, w0e_, b0, w1, b1, w2, b2, g, beta, o_ref):
        e = e_ref[...]
        h = jnp.maximum(_dot(e, w0e_[...]) + g_ref[...] + b0[...], 0.0)
        h = jnp.maximum(_dot(h, w1[...]) + b1[...], 0.0)
        h = _dot(h, w2[...]) + b2[...]
        o_ref[...] = e + _ln(h, g[...], beta[...])

    args = (edge_lat, gsum, w0e, p['b0'].reshape(1, -1),
            p['W1'], p['b1'].reshape(1, -1), p['W2'], p['b2'].reshape(1, -1),
            p['g'].reshape(1, -1), p['beta'].reshape(1, -1))
    row_spec = pl.BlockSpec((block, LAT), lambda i: (i, 0))
    in_specs = [row_spec, row_spec]
    in_specs += [_full(a.shape) for a in args[2:]]
    return pl.pallas_call(
        body,
        grid=(NE // block,),
        in_specs=in_specs,
        out_specs=row_spec,
        out_shape=jax.ShapeDtypeStruct((NE, LAT), jnp.float32),
    )(*args)


def _node_step(node_lat, agg2, p, proj=None, block=2000):
    w0n = p['W0'][0:LAT]
    w0a = p['W0'][LAT:2 * LAT]

    def body(n_ref, a_ref, w0n_, w0a_, b0, w1, b1, w2, b2, g, beta, *rest):
        nl = n_ref[...]
        agg = a_ref[0] + a_ref[1]
        h = _dot(nl, w0n_[...]) + _dot(agg, w0a_[...]) + b0[...]
        h = jnp.maximum(h, 0.0)
        h = jnp.maximum(_dot(h, w1[...]) + b1[...], 0.0)
        h = _dot(h, w2[...]) + b2[...]
        y = nl + _ln(h, g[...], beta[...])
        if proj is None:
            rest[-1][...] = y
        else:
            ws, wr, o_ref, sp_ref, rp_ref = rest
            o_ref[...] = y
            sp_ref[...] = _dot(y, ws[...])
            rp_ref[...] = _dot(y, wr[...])

    args = [node_lat, agg2, w0n, w0a, p['b0'].reshape(1, -1), p['W1'],
            p['b1'].reshape(1, -1), p['W2'], p['b2'].reshape(1, -1),
            p['g'].reshape(1, -1), p['beta'].reshape(1, -1)]
    if proj is not None:
        args += [proj[0], proj[1]]
    in_specs = [pl.BlockSpec((block, LAT), lambda i: (i, 0)),
                pl.BlockSpec((NCORE, block, LAT), lambda i: (0, i, 0))]
    in_specs += [_full(a.shape) for a in args[2:]]
    row_spec = pl.BlockSpec((block, LAT), lambda i: (i, 0))
    n_out = 1 if proj is None else 3
    return pl.pallas_call(
        body,
        grid=(NN // block,),
        in_specs=in_specs,
        out_specs=[row_spec] * n_out if proj else row_spec,
        out_shape=([jax.ShapeDtypeStruct((NN, LAT), jnp.float32)] * n_out
                   if proj else jax.ShapeDtypeStruct((NN, LAT), jnp.float32)),
    )(*args)


def _decode(node_lat, p, block=2000):
    def body(x_ref, w0, b0, w1, b1, w2, b2, o_ref):
        h = jnp.maximum(_dot(x_ref[...], w0[...]) + b0[...], 0.0)
        h = jnp.maximum(_dot(h, w1[...]) + b1[...], 0.0)
        o_ref[...] = _dot(h, w2[...]) + b2[...]

    dout = p['W2'].shape[1]
    args = (node_lat, p['W0'], p['b0'].reshape(1, -1), p['W1'],
            p['b1'].reshape(1, -1), p['W2'], p['b2'].reshape(1, -1))
    in_specs = [pl.BlockSpec((block, LAT), lambda i: (i, 0))]
    in_specs += [_full(a.shape) for a in args[1:]]
    return pl.pallas_call(
        body,
        grid=(NN // block,),
        in_specs=in_specs,
        out_specs=pl.BlockSpec((block, dout), lambda i: (i, 0)),
        out_shape=jax.ShapeDtypeStruct((NN, dout), jnp.float32),
    )(*args)


def _w_sr(step_params):
    w0 = step_params['edge']['W0']
    return w0[LAT:2 * LAT], w0[2 * LAT:3 * LAT]


def kernel(node_features, edge_features, senders, receivers, params):
    steps = params['steps']
    node_lat, sp, rp = _enc_call(node_features, params['node_enc'],
                                 block=2000, proj=_w_sr(steps[0]))
    edge_lat = _enc_call(edge_features, params['edge_enc'], block=2560)
    recv3 = receivers.reshape(NW, NCH, CH)
    zeros = jnp.zeros((NNP, LAT), jnp.float32)
    for i, step in enumerate(steps):
        gsum = _sc_gather_sum(sp, rp, senders, receivers)
        edge_lat = _edge_step(edge_lat, gsum, step['edge'])
        agg2 = _sc_scatter(edge_lat, recv3, zeros)
        if i + 1 < len(steps):
            node_lat, sp, rp = _node_step(node_lat, agg2, step['node'],
                                          proj=_w_sr(steps[i + 1]))
        else:
            node_lat = _node_step(node_lat, agg2, step['node'])
    return _decode(node_lat, params['decoder'])


# speedup vs baseline: 5.2011x; 1.0965x over previous
import functools

import jax
import jax.numpy as jnp
from jax import lax
from jax.experimental import pallas as pl
from jax.experimental.pallas import tpu as pltpu
from jax.experimental.pallas import tpu_sc as plsc

LAT = 128
NN = 10000
NE = 320000
NEH = NE // 2
NCORE = 2
NSUB = 16
NW = NCORE * NSUB
EPW = NEH // NW
CH = 40
NCH = EPW // CH
NSLOT = 5
NSLOT_SC = 5
NNP = 10240
NPS = NNP // NSUB


def _sc_gather_sum(sp, rp, snd_h, rcv_h):
    mesh = plsc.VectorSubcoreMesh(core_axis_name="c", subcore_axis_name="s")
    scr = [pltpu.VMEM((EPW,), jnp.int32), pltpu.VMEM((EPW,), jnp.int32)]
    scr += [pltpu.VMEM((CH, LAT), jnp.float32) for _ in range(2 * NSLOT)]
    scr += [pltpu.SemaphoreType.DMA for _ in range(2 * NSLOT + 1)]

    @functools.partial(
        pl.kernel,
        mesh=mesh,
        out_type=jax.ShapeDtypeStruct((NEH, LAT), jnp.float32),
        scratch_types=scr,
    )
    def k(sp_hbm, rp_hbm, snd_hbm, rcv_hbm, out, idx_s, idx_r, *rest):
        bufs = rest[:2 * NSLOT]
        sems = rest[2 * NSLOT:]
        bs = bufs[0::2]
        br = bufs[1::2]
        sg = sems[:NSLOT]
        rg = sems[NSLOT:2 * NSLOT]
        ws = sems[2 * NSLOT]
        wid = lax.axis_index("c") * NSUB + lax.axis_index("s")
        base = wid * EPW
        pltpu.sync_copy(snd_hbm.at[pl.ds(base, EPW)], idx_s)
        pltpu.sync_copy(rcv_hbm.at[pl.ds(base, EPW)], idx_r)

        def body(kk, carry):
            j0 = kk * NSLOT
            gs, gr = [], []
            for t in range(NSLOT):
                off = (j0 + t) * CH
                gs.append(pltpu.async_copy(
                    sp_hbm.at[idx_s.at[pl.ds(off, CH)]], bs[t], sg[t]))
                gr.append(pltpu.async_copy(
                    rp_hbm.at[idx_r.at[pl.ds(off, CH)]], br[t], rg[t]))
            wcs = []
            for t in range(NSLOT):
                gs[t].wait()
                gr[t].wait()

                def add_row(r, _, t=t):
                    for cc in range(LAT // 16):
                        sl = pl.ds(cc * 16, 16)
                        bs[t][r, sl] = bs[t][r, sl] + br[t][r, sl]
                    return 0

                lax.fori_loop(0, CH, add_row, 0)
                wcs.append(pltpu.async_copy(
                    bs[t], out.at[pl.ds(base + (j0 + t) * CH, CH)], ws))
            for w in wcs:
                w.wait()
            return carry

        lax.fori_loop(0, NCH // NSLOT, body, 0)

    return k(sp, rp, snd_h, rcv_h)


def _sc_scatter(edge_h, recv3_h, zeros):
    mesh = plsc.VectorSubcoreMesh(core_axis_name="c", subcore_axis_name="s")
    scr = [pltpu.VMEM((NCH, CH), jnp.int32)]
    scr += [pltpu.VMEM((CH, LAT), jnp.float32) for _ in range(NSLOT_SC)]
    scr += [pltpu.VMEM_SHARED((NNP, LAT), jnp.float32)]
    scr += [pltpu.SemaphoreType.DMA for _ in range(NSLOT_SC + 1)]

    @functools.partial(
        pl.kernel,
        mesh=mesh,
        out_type=jax.ShapeDtypeStruct((NCORE, NNP, LAT), jnp.float32),
        scratch_types=scr,
    )
    def k(edge_hbm, idx_hbm, zero_hbm, out, idxv, *rest):
        rows = rest[:NSLOT_SC]
        agg = rest[NSLOT_SC]
        sems = rest[NSLOT_SC + 1:]
        ld = sems[:NSLOT_SC]
        sc = sems[NSLOT_SC]
        c = lax.axis_index("c")
        s = lax.axis_index("s")
        wid = c * NSUB + s
        base = wid * EPW
        pltpu.sync_copy(idx_hbm.at[wid], idxv)
        pltpu.sync_copy(zero_hbm.at[pl.ds(s * NPS, NPS)],
                        agg.at[pl.ds(s * NPS, NPS)])
        plsc.subcore_barrier()

        def body(kk, carry):
            j0 = kk * NSLOT_SC
            lds = [pltpu.async_copy(
                edge_hbm.at[pl.ds(base + (j0 + t) * CH, CH)], rows[t], ld[t])
                for t in range(NSLOT_SC)]
            scs = []
            for t in range(NSLOT_SC):
                lds[t].wait()
                scs.append(pltpu.async_copy(
                    rows[t], agg.at[idxv.at[j0 + t]], sc, add=True))
            for x in scs:
                x.wait()
            return carry

        lax.fori_loop(0, NCH // NSLOT_SC, body, 0)
        plsc.subcore_barrier()
        pltpu.sync_copy(agg.at[pl.ds(s * NPS, NPS)],
                        out.at[c, pl.ds(s * NPS, NPS)])

    return k(edge_h, recv3_h, zeros)


def _ln(h, g, beta):
    mu = jnp.mean(h, axis=-1, keepdims=True)
    d = h - mu
    var = jnp.mean(d * d, axis=-1, keepdims=True)
    return d * lax.rsqrt(var + 1e-5) * g + beta


def _dot(a, b):
    return jnp.dot(a, b, preferred_element_type=jnp.float32)


def _full(shape):
    nd = len(shape)
    return pl.BlockSpec(shape, lambda i: (0,) * nd)


def _enc_call(x, p, block, proj=None):
    n, din = x.shape

    def body(x_ref, w0, b0, w1, b1, w2, b2, g, beta, *rest):
        h = jnp.maximum(_dot(x_ref[...], w0[...]) + b0[...], 0.0)
        h = jnp.maximum(_dot(h, w1[...]) + b1[...], 0.0)
        h = _dot(h, w2[...]) + b2[...]
        y = _ln(h, g[...], beta[...])
        if proj is None:
            rest[-1][...] = y
        else:
            ws, wr, o_ref, sp_ref, rp_ref = rest
            o_ref[...] = y
            sp_ref[...] = _dot(y, ws[...])
            rp_ref[...] = _dot(y, wr[...])

    args = [x, p['W0'], p['b0'].reshape(1, -1), p['W1'],
            p['b1'].reshape(1, -1), p['W2'], p['b2'].reshape(1, -1),
            p['g'].reshape(1, -1), p['beta'].reshape(1, -1)]
    if proj is not None:
        args += [proj[0], proj[1]]
    in_specs = [pl.BlockSpec((block, din), lambda i: (i, 0))]
    in_specs += [_full(a.shape) for a in args[1:]]
    row_spec = pl.BlockSpec((block, LAT), lambda i: (i, 0))
    n_out = 1 if proj is None else 3
    return pl.pallas_call(
        body,
        grid=(n // block,),
        in_specs=in_specs,
        out_specs=[row_spec] * n_out if proj else row_spec,
        out_shape=([jax.ShapeDtypeStruct((n, LAT), jnp.float32)] * n_out
                   if proj else jax.ShapeDtypeStruct((n, LAT), jnp.float32)),
    )(*args)


def _edge_step(edge_h, gsum_h, p, block=2000):
    w0e = p['W0'][0:LAT]

    def body(e_ref, g_ref, w0e_, b0, w1, b1, w2, b2, g, beta, o_ref):
        e = e_ref[...]
        h = jnp.maximum(_dot(e, w0e_[...]) + g_ref[...] + b0[...], 0.0)
        h = jnp.maximum(_dot(h, w1[...]) + b1[...], 0.0)
        h = _dot(h, w2[...]) + b2[...]
        o_ref[...] = e + _ln(h, g[...], beta[...])

    args = (edge_h, gsum_h, w0e, p['b0'].reshape(1, -1),
            p['W1'], p['b1'].reshape(1, -1), p['W2'], p['b2'].reshape(1, -1),
            p['g'].reshape(1, -1), p['beta'].reshape(1, -1))
    row_spec = pl.BlockSpec((block, LAT), lambda i: (i, 0))
    in_specs = [row_spec, row_spec]
    in_specs += [_full(a.shape) for a in args[2:]]
    return pl.pallas_call(
        body,
        grid=(NEH // block,),
        in_specs=in_specs,
        out_specs=row_spec,
        out_shape=jax.ShapeDtypeStruct((NEH, LAT), jnp.float32),
    )(*args)


def _node_step(node_lat, agg_a, agg_b, p, proj=None, block=2000):
    w0n = p['W0'][0:LAT]
    w0a = p['W0'][LAT:2 * LAT]

    def body(n_ref, a_ref, b_ref, w0n_, w0a_, b0, w1, b1, w2, b2, g, beta,
             *rest):
        nl = n_ref[...]
        agg = (a_ref[0] + a_ref[1]) + (b_ref[0] + b_ref[1])
        h = _dot(nl, w0n_[...]) + _dot(agg, w0a_[...]) + b0[...]
        h = jnp.maximum(h, 0.0)
        h = jnp.maximum(_dot(h, w1[...]) + b1[...], 0.0)
        h = _dot(h, w2[...]) + b2[...]
        y = nl + _ln(h, g[...], beta[...])
        if proj is None:
            rest[-1][...] = y
        else:
            ws, wr, o_ref, sp_ref, rp_ref = rest
            o_ref[...] = y
            sp_ref[...] = _dot(y, ws[...])
            rp_ref[...] = _dot(y, wr[...])

    args = [node_lat, agg_a, agg_b, w0n, w0a, p['b0'].reshape(1, -1),
            p['W1'], p['b1'].reshape(1, -1), p['W2'], p['b2'].reshape(1, -1),
            p['g'].reshape(1, -1), p['beta'].reshape(1, -1)]
    if proj is not None:
        args += [proj[0], proj[1]]
    agg_spec = pl.BlockSpec((NCORE, block, LAT), lambda i: (0, i, 0))
    in_specs = [pl.BlockSpec((block, LAT), lambda i: (i, 0)),
                agg_spec, agg_spec]
    in_specs += [_full(a.shape) for a in args[3:]]
    row_spec = pl.BlockSpec((block, LAT), lambda i: (i, 0))
    n_out = 1 if proj is None else 3
    return pl.pallas_call(
        body,
        grid=(NN // block,),
        in_specs=in_specs,
        out_specs=[row_spec] * n_out if proj else row_spec,
        out_shape=([jax.ShapeDtypeStruct((NN, LAT), jnp.float32)] * n_out
                   if proj else jax.ShapeDtypeStruct((NN, LAT), jnp.float32)),
    )(*args)


def _decode(node_lat, p, block=2000):
    def body(x_ref, w0, b0, w1, b1, w2, b2, o_ref):
        h = jnp.maximum(_dot(x_ref[...], w0[...]) + b0[...], 0.0)
        h = jnp.maximum(_dot(h, w1[...]) + b1[...], 0.0)
        o_ref[...] = _dot(h, w2[...]) + b2[...]

    dout = p['W2'].shape[1]
    args = (node_lat, p['W0'], p['b0'].reshape(1, -1), p['W1'],
            p['b1'].reshape(1, -1), p['W2'], p['b2'].reshape(1, -1))
    in_specs = [pl.BlockSpec((block, LAT), lambda i: (i, 0))]
    in_specs += [_full(a.shape) for a in args[1:]]
    return pl.pallas_call(
        body,
        grid=(NN // block,),
        in_specs=in_specs,
        out_specs=pl.BlockSpec((block, dout), lambda i: (i, 0)),
        out_shape=jax.ShapeDtypeStruct((NN, dout), jnp.float32),
    )(*args)


def _w_sr(step_params):
    w0 = step_params['edge']['W0']
    return w0[LAT:2 * LAT], w0[2 * LAT:3 * LAT]


def kernel(node_features, edge_features, senders, receivers, params):
    steps = params['steps']
    node_lat, sp, rp = _enc_call(node_features, params['node_enc'],
                                 block=2000, proj=_w_sr(steps[0]))
    snd = (senders[:NEH], senders[NEH:])
    rcv = (receivers[:NEH], receivers[NEH:])
    rcv3 = tuple(r.reshape(NW, NCH, CH) for r in rcv)
    e_lat = [_enc_call(edge_features[:NEH], params['edge_enc'], block=2000),
             _enc_call(edge_features[NEH:], params['edge_enc'], block=2000)]
    zeros = jnp.zeros((NNP, LAT), jnp.float32)
    for i, step in enumerate(steps):
        gsum = [_sc_gather_sum(sp, rp, snd[h], rcv[h]) for h in range(2)]
        e_lat = [_edge_step(e_lat[h], gsum[h], step['edge'])
                 for h in range(2)]
        aggs = [_sc_scatter(e_lat[h], rcv3[h], zeros) for h in range(2)]
        if i + 1 < len(steps):
            node_lat, sp, rp = _node_step(node_lat, aggs[0], aggs[1],
                                          step['node'],
                                          proj=_w_sr(steps[i + 1]))
        else:
            node_lat = _node_step(node_lat, aggs[0], aggs[1], step['node'])
    return _decode(node_lat, params['decoder'])
